# Initial kernel scaffold; baseline (speedup 1.0000x reference)
import jax
import jax.numpy as jnp
from jax.experimental import pallas as pl


def kernel(h0, h1, h2, index):
    raise NotImplementedError("write your pallas kernel here")


# speedup vs baseline: 2.0016x; 2.0016x over previous
import jax
import jax.numpy as jnp
from jax.experimental import pallas as pl

_NSEG = 1024
_N = 100000
_B = 200
_GRID = _N // _B


def _body(idx_ref, h0_ref, h1_ref, h2_ref, out_ref):
    pid = pl.program_id(0)

    @pl.when(pid == 0)
    def _init():
        out_ref[...] = jnp.full(out_ref.shape, -jnp.inf, jnp.float32)

    idx = idx_ref[0]
    smin = idx[0, 0]
    smax = idx[_B - 1, 0]
    hs = (h0_ref[...], h1_ref[...], h2_ref[...])

    def seg_body(i, _):
        s = smin + i
        m = idx == s
        base = (s // 8) * 8
        rowmask = jax.lax.broadcasted_iota(jnp.int32, (8, 1), 0) == (s - base)
        vs = [
            jnp.max(jnp.where(m, hk, -jnp.inf), axis=0, keepdims=True)
            for hk in hs
        ]
        v = jnp.concatenate(vs, axis=1)
        win = out_ref[pl.ds(base, 8), :]
        upd = jnp.where(rowmask, jnp.broadcast_to(v, (8, 384)), -jnp.inf)
        out_ref[pl.ds(base, 8), :] = jnp.maximum(win, upd)
        return 0

    jax.lax.fori_loop(0, smax - smin + 1, seg_body, 0)


def kernel(h0, h1, h2, index):
    idx3 = index.astype(jnp.int32).reshape(_GRID, _B, 1)
    return pl.pallas_call(
        _body,
        grid=(_GRID,),
        in_specs=[
            pl.BlockSpec((1, _B, 1), lambda i: (i, 0, 0)),
            pl.BlockSpec((_B, 128), lambda i: (i, 0)),
            pl.BlockSpec((_B, 128), lambda i: (i, 0)),
            pl.BlockSpec((_B, 128), lambda i: (i, 0)),
        ],
        out_specs=pl.BlockSpec((_NSEG, 384), lambda i: (0, 0)),
        out_shape=jax.ShapeDtypeStruct((_NSEG, 384), jnp.float32),
    )(idx3, h0, h1, h2)


# speedup vs baseline: 3.4800x; 1.7386x over previous
import jax
import jax.numpy as jnp
from jax import lax
from jax.experimental import pallas as pl
from jax.experimental.pallas import tpu as pltpu
from jax.experimental.pallas import tpu_sc as plsc

_NSEG = 1024
_N = 100000
_NW = 32
_SPW = _NSEG // _NW
_CR = 448


def _sc_body(h0, h1, h2, starts_hbm, out_hbm, starts_v, buf, res, sem0, sem1):
    nc = 2
    wid = lax.axis_index("s") * nc + lax.axis_index("c")
    base = wid * _SPW

    pltpu.sync_copy(starts_hbm.at[pl.ds(base, 64)], starts_v)

    def sval(k):
        return starts_v[pl.ds(k, 16)][0]

    lo = sval(0)
    hi = sval(_SPW)
    lo_a = (lo // 8) * 8

    neg = jnp.full((16,), -jnp.inf, jnp.float32)

    def init_body(s, _):
        for j in range(24):
            res[s, pl.ds(16 * j, 16)] = neg
        return 0

    lax.fori_loop(0, _SPW, init_body, 0)

    nch = (hi - lo_a + _CR - 1) // _CR
    npairs = (nch + 1) // 2

    def chunk_start(k):
        return jnp.minimum(lo_a + k * _CR, _N - _CR)

    for arr, h in enumerate((h0, h1, h2)):
        coff = 128 * arr

        def issue(k, parity, sem, _h=h):
            pltpu.async_copy(
                _h.at[pl.ds(chunk_start(k), _CR), :],
                buf.at[pl.ds(parity * _CR, _CR), :],
                sem,
            )

        def wait_chunk(parity, sem, _h=h):
            pltpu.make_async_copy(
                _h.at[pl.ds(0, _CR), :],
                buf.at[pl.ds(parity * _CR, _CR), :],
                sem,
            ).wait()

        def process(k, parity, _coff=coff):
            cb = chunk_start(k)
            c1 = cb + _CR
            soff = parity * _CR - 0

            def seg_body(s, _):
                a = jnp.maximum(sval(s), cb)
                b = jnp.minimum(sval(s + 1), c1)
                regs = tuple(
                    res[s, pl.ds(_coff + 16 * j, 16)] for j in range(8))

                def rbody(rr, rg):
                    row = soff + (rr - cb)
                    return tuple(
                        jnp.maximum(rg[j], buf[row, pl.ds(16 * j, 16)])
                        for j in range(8)
                    )

                regs = lax.fori_loop(a, b, rbody, regs)
                for j in range(8):
                    res[s, pl.ds(_coff + 16 * j, 16)] = regs[j]
                return 0

            lax.fori_loop(0, _SPW, seg_body, 0)

        issue(jnp.int32(0), 0, sem0)
        issue(jnp.int32(1), 1, sem1)

        def pair_body(p, _):
            k0 = 2 * p
            wait_chunk(0, sem0)
            process(k0, 0)
            issue(k0 + 2, 0, sem0)
            wait_chunk(1, sem1)
            process(k0 + 1, 1)
            issue(k0 + 3, 1, sem1)
            return 0

        lax.fori_loop(0, npairs, pair_body, 0)

        wait_chunk(0, sem0)
        wait_chunk(1, sem1)

    pltpu.sync_copy(res, out_hbm.at[pl.ds(base, _SPW), :])


def kernel(h0, h1, h2, index):
    idx32 = index.astype(jnp.int32)
    targets = jnp.arange(_NSEG + 1, dtype=jnp.int32)
    starts = jnp.searchsorted(idx32, targets).astype(jnp.int32)
    starts = jnp.concatenate(
        [starts, jnp.full((39,), jnp.int32(_N))])

    mesh = plsc.VectorSubcoreMesh(
        core_axis_name="c", subcore_axis_name="s", num_cores=2, num_subcores=16)
    f = pl.kernel(
        _sc_body,
        out_type=jax.ShapeDtypeStruct((_NSEG, 384), jnp.float32),
        mesh=mesh,
        scratch_types=[
            pltpu.VMEM((64,), jnp.int32),
            pltpu.VMEM((2 * _CR, 128), jnp.float32),
            pltpu.VMEM((_SPW, 384), jnp.float32),
            pltpu.SemaphoreType.DMA,
            pltpu.SemaphoreType.DMA,
        ],
    )
    return f(h0, h1, h2, starts)


# speedup vs baseline: 4.6977x; 1.3499x over previous
import jax
import jax.numpy as jnp
from jax import lax
from jax.experimental import pallas as pl
from jax.experimental.pallas import tpu as pltpu
from jax.experimental.pallas import tpu_sc as plsc

_NSEG = 1024
_N = 100000
_NW = 32
_SPW = _NSEG // _NW
_CR = 448


def _sc_body(h0, h1, h2, starts_hbm, out_hbm, starts_v, buf, res, sem0, sem1):
    nc = 2
    wid = lax.axis_index("s") * nc + lax.axis_index("c")
    base = wid * _SPW

    pltpu.sync_copy(starts_hbm.at[pl.ds(base, 64)], starts_v)

    def sval(k):
        return starts_v[pl.ds(k, 16)][0]

    lo = sval(0)
    hi = sval(_SPW)
    lo_a = (lo // 8) * 8

    neg = jnp.full((16,), -jnp.inf, jnp.float32)

    def init_body(s, _):
        for j in range(24):
            res[s, pl.ds(16 * j, 16)] = neg
        return 0

    lax.fori_loop(0, _SPW, init_body, 0)

    nch = (hi - lo_a + _CR - 1) // _CR
    npairs = (nch + 1) // 2

    def chunk_start(k):
        return jnp.minimum(lo_a + k * _CR, _N - _CR)

    for arr, h in enumerate((h0, h1, h2)):
        coff = 128 * arr

        def issue(k, parity, sem, _h=h):
            pltpu.async_copy(
                _h.at[pl.ds(chunk_start(k), _CR), :],
                buf.at[pl.ds(parity * _CR, _CR), :],
                sem,
            )

        def wait_chunk(parity, sem, _h=h):
            pltpu.make_async_copy(
                _h.at[pl.ds(0, _CR), :],
                buf.at[pl.ds(parity * _CR, _CR), :],
                sem,
            ).wait()

        def process(k, parity, _coff=coff):
            cb = chunk_start(k)
            c1 = cb + _CR
            soff = parity * _CR - 0

            def seg_body(s, _):
                a = jnp.maximum(sval(s), cb)
                b = jnp.minimum(sval(s + 1), c1)
                regs = tuple(
                    res[s, pl.ds(_coff + 16 * j, 16)] for j in range(8))

                def rbody(rr, rg):
                    row = soff + (rr - cb)
                    return tuple(
                        jnp.maximum(rg[j], buf[row, pl.ds(16 * j, 16)])
                        for j in range(8)
                    )

                regs = lax.fori_loop(a, b, rbody, regs)
                for j in range(8):
                    res[s, pl.ds(_coff + 16 * j, 16)] = regs[j]
                return 0

            lax.fori_loop(0, _SPW, seg_body, 0)

        issue(jnp.int32(0), 0, sem0)
        issue(jnp.int32(1), 1, sem1)

        def pair_body(p, _):
            k0 = 2 * p
            wait_chunk(0, sem0)
            process(k0, 0)
            issue(k0 + 2, 0, sem0)
            wait_chunk(1, sem1)
            process(k0 + 1, 1)
            issue(k0 + 3, 1, sem1)
            return 0

        lax.fori_loop(0, npairs, pair_body, 0)

        wait_chunk(0, sem0)
        wait_chunk(1, sem1)

    pltpu.sync_copy(res, out_hbm.at[pl.ds(base, _SPW), :])


def kernel(h0, h1, h2, index):
    idx32 = index.astype(jnp.int32)
    targets = jnp.arange(_NSEG + 1, dtype=jnp.int32)
    starts = jnp.searchsorted(
        idx32, targets, method="compare_all").astype(jnp.int32)
    starts = jnp.concatenate(
        [starts, jnp.full((39,), jnp.int32(_N))])

    mesh = plsc.VectorSubcoreMesh(
        core_axis_name="c", subcore_axis_name="s", num_cores=2, num_subcores=16)
    f = pl.kernel(
        _sc_body,
        out_type=jax.ShapeDtypeStruct((_NSEG, 384), jnp.float32),
        mesh=mesh,
        scratch_types=[
            pltpu.VMEM((64,), jnp.int32),
            pltpu.VMEM((2 * _CR, 128), jnp.float32),
            pltpu.VMEM((_SPW, 384), jnp.float32),
            pltpu.SemaphoreType.DMA,
            pltpu.SemaphoreType.DMA,
        ],
    )
    return f(h0, h1, h2, starts)


# speedup vs baseline: 6.7088x; 1.4281x over previous
import jax
import jax.numpy as jnp
from jax import lax
from jax.experimental import pallas as pl
from jax.experimental.pallas import tpu as pltpu
from jax.experimental.pallas import tpu_sc as plsc

_NSEG = 1024
_N = 100000
_NW = 32
_SPW = _NSEG // _NW
_CR = 448


def _sc_body(h0, h1, h2, starts_hbm, out_hbm, starts_v, buf, res, sem0, sem1):
    nc = 2
    wid = lax.axis_index("s") * nc + lax.axis_index("c")
    base = wid * _SPW

    pltpu.sync_copy(starts_hbm.at[pl.ds(base, 64)], starts_v)

    def sval(k):
        return starts_v[pl.ds(k, 16)][0]

    lo = sval(0)
    hi = sval(_SPW)
    lo_a = (lo // 8) * 8

    neg = jnp.full((16,), -jnp.inf, jnp.float32)

    def init_body(s, _):
        for j in range(24):
            res[s, pl.ds(16 * j, 16)] = neg
        return 0

    lax.fori_loop(0, _SPW, init_body, 0)

    nch = (hi - lo_a + _CR - 1) // _CR
    npairs = (nch + 1) // 2

    def chunk_start(k):
        return jnp.minimum(lo_a + k * _CR, _N - _CR)

    for arr, h in enumerate((h0, h1, h2)):
        coff = 128 * arr

        def issue(k, parity, sem, _h=h):
            pltpu.async_copy(
                _h.at[pl.ds(chunk_start(k), _CR), :],
                buf.at[pl.ds(parity * _CR, _CR), :],
                sem,
            )

        def wait_chunk(parity, sem, _h=h):
            pltpu.make_async_copy(
                _h.at[pl.ds(0, _CR), :],
                buf.at[pl.ds(parity * _CR, _CR), :],
                sem,
            ).wait()

        def process(k, parity, _coff=coff):
            cb = chunk_start(k)
            c1 = cb + _CR
            soff = parity * _CR - 0

            def seg_body(s, _):
                a = jnp.maximum(sval(s), cb)
                b = jnp.minimum(sval(s + 1), c1)
                regs = tuple(
                    res[s, pl.ds(_coff + 16 * j, 16)] for j in range(8))

                def rbody(rr, rg):
                    row = soff + (rr - cb)
                    return tuple(
                        jnp.maximum(rg[j], buf[row, pl.ds(16 * j, 16)])
                        for j in range(8)
                    )

                regs = lax.fori_loop(a, b, rbody, regs)
                for j in range(8):
                    res[s, pl.ds(_coff + 16 * j, 16)] = regs[j]
                return 0

            lax.fori_loop(0, _SPW, seg_body, 0)

        issue(jnp.int32(0), 0, sem0)
        issue(jnp.int32(1), 1, sem1)

        def pair_body(p, _):
            k0 = 2 * p
            wait_chunk(0, sem0)
            process(k0, 0)
            issue(k0 + 2, 0, sem0)
            wait_chunk(1, sem1)
            process(k0 + 1, 1)
            issue(k0 + 3, 1, sem1)
            return 0

        lax.fori_loop(0, npairs, pair_body, 0)

        wait_chunk(0, sem0)
        wait_chunk(1, sem1)

    pltpu.sync_copy(res, out_hbm.at[pl.ds(base, _SPW), :])


def kernel(h0, h1, h2, index):
    idx32 = index.astype(jnp.int32)
    targets = jnp.arange(_NSEG + 1, dtype=jnp.int32)
    blk = idx32.reshape(1000, 100)
    mins = blk[:, 0]
    nb = jnp.sum((mins[None, :] < targets[:, None]), axis=1, dtype=jnp.int32)
    b = jnp.maximum(nb - 1, 0)
    rows = blk[b]
    within = jnp.sum(rows < targets[:, None], axis=1, dtype=jnp.int32)
    starts = (100 * b + within).astype(jnp.int32)
    starts = jnp.concatenate(
        [starts, jnp.full((39,), jnp.int32(_N))])

    mesh = plsc.VectorSubcoreMesh(
        core_axis_name="c", subcore_axis_name="s", num_cores=2, num_subcores=16)
    f = pl.kernel(
        _sc_body,
        out_type=jax.ShapeDtypeStruct((_NSEG, 384), jnp.float32),
        mesh=mesh,
        scratch_types=[
            pltpu.VMEM((64,), jnp.int32),
            pltpu.VMEM((2 * _CR, 128), jnp.float32),
            pltpu.VMEM((_SPW, 384), jnp.float32),
            pltpu.SemaphoreType.DMA,
            pltpu.SemaphoreType.DMA,
        ],
    )
    return f(h0, h1, h2, starts)
